# Initial kernel scaffold; baseline (speedup 1.0000x reference)
import jax
import jax.numpy as jnp
from jax.experimental import pallas as pl


def kernel(x, edge_index, edge_attr, graph_ids, params):
    raise NotImplementedError("write your pallas kernel here")


# speedup vs baseline: 1.1078x; 1.1078x over previous
import functools

import jax
import jax.numpy as jnp
from jax import lax
from jax.experimental import pallas as pl
from jax.experimental.pallas import tpu as pltpu
from jax.experimental.pallas import tpu_sc as plsc

N_NODES = 10000
N_EDGES = 160000
N_GRAPHS = 256
D_ATOM = 30
D_BOND = 11
D_NODE = 32
D_EH = 128
D_EOUT = 32
DIM = D_NODE + D_EOUT
STEPS = 3
ITERS = 6
LAYERS = 3
N_TASKS = 12
N_CLASSES = 2

SC_CORES = 2
SC_SUBCORES = 16
N_WORKERS = SC_CORES * SC_SUBCORES
CHUNK = 128
N_CHUNKS = N_EDGES // CHUNK
FULL_ROUNDS = N_CHUNKS // N_WORKERS
TAIL = N_CHUNKS - FULL_ROUNDS * N_WORKERS
ZROWS = N_NODES // SC_SUBCORES

@functools.cache
def _mesh():
    return plsc.VectorSubcoreMesh(
        core_axis_name="c", subcore_axis_name="s",
        num_cores=SC_CORES, num_subcores=SC_SUBCORES)

_HIGH = jax.lax.Precision.HIGHEST


def _dot(a, b):
    return jnp.dot(a, b, precision=_HIGH, preferred_element_type=jnp.float32)


def _wid():
    return lax.axis_index("s") * SC_CORES + lax.axis_index("c")


def _sc_gather_body(table, idx, out, idx_v, rows_v):
    w = _wid()

    def do_chunk(c):
        pltpu.sync_copy(idx.at[pl.ds(c * CHUNK, CHUNK)], idx_v)
        pltpu.sync_copy(table.at[idx_v], rows_v)
        pltpu.sync_copy(rows_v, out.at[pl.ds(c * CHUNK, CHUNK)])

    def body(k, carry):
        do_chunk(w + k * N_WORKERS)
        return carry

    lax.fori_loop(0, FULL_ROUNDS, body, 0, unroll=False)

    @pl.when(w < TAIL)
    def _tail():
        do_chunk(FULL_ROUNDS * N_WORKERS + w)


@jax.jit
def sc_gather(table, idx):
    return pl.kernel(
        _sc_gather_body,
        out_type=jax.ShapeDtypeStruct((N_EDGES, D_NODE), jnp.float32),
        mesh=_mesh(),
        compiler_params=pltpu.CompilerParams(use_tc_tiling_on_sc=False),
        scratch_types=[
            pltpu.VMEM((CHUNK,), jnp.int32),
            pltpu.VMEM((CHUNK, D_NODE), jnp.float32),
        ],
    )(table, idx)


def _sc_scatter_body(vals, idx, zeros, out, idx_v, v_v, acc):
    w = _wid()
    sid = lax.axis_index("s")
    cid = lax.axis_index("c")

    pltpu.sync_copy(zeros, acc.at[pl.ds(sid * ZROWS, ZROWS)])
    plsc.subcore_barrier()

    def do_chunk(c):
        pltpu.sync_copy(idx.at[pl.ds(c * CHUNK, CHUNK)], idx_v)
        pltpu.sync_copy(vals.at[pl.ds(c * CHUNK, CHUNK)], v_v)
        pltpu.sync_copy(v_v, acc.at[idx_v], add=True)

    def body(k, carry):
        do_chunk(w + k * N_WORKERS)
        return carry

    lax.fori_loop(0, FULL_ROUNDS, body, 0, unroll=False)

    @pl.when(w < TAIL)
    def _tail():
        do_chunk(FULL_ROUNDS * N_WORKERS + w)
    plsc.subcore_barrier()

    pltpu.sync_copy(
        acc.at[pl.ds(sid * ZROWS, ZROWS)],
        out.at[pl.ds(cid * N_NODES + sid * ZROWS, ZROWS)])


@jax.jit
def sc_scatter_add(vals, idx, zeros):
    return pl.kernel(
        _sc_scatter_body,
        out_type=jax.ShapeDtypeStruct((SC_CORES * N_NODES, D_NODE), jnp.float32),
        mesh=_mesh(),
        compiler_params=pltpu.CompilerParams(use_tc_tiling_on_sc=False),
        scratch_types=[
            pltpu.VMEM((CHUNK,), jnp.int32),
            pltpu.VMEM((CHUNK, D_NODE), jnp.float32),
            pltpu.VMEM_SHARED((N_NODES, D_NODE), jnp.float32),
        ],
    )(vals, idx, zeros)


def _sc_feat_body(node_emb, edge_emb, src, dst, zeros, out1, out2,
                  s_v, d_v, g_v, e_v, acc1, acc2):
    w = _wid()
    sid = lax.axis_index("s")
    cid = lax.axis_index("c")

    pltpu.sync_copy(zeros, acc1.at[pl.ds(sid * ZROWS, ZROWS)])
    pltpu.sync_copy(zeros, acc2.at[pl.ds(sid * ZROWS, ZROWS)])
    plsc.subcore_barrier()

    def do_chunk(c):
        pltpu.sync_copy(src.at[pl.ds(c * CHUNK, CHUNK)], s_v)
        pltpu.sync_copy(dst.at[pl.ds(c * CHUNK, CHUNK)], d_v)
        pltpu.sync_copy(node_emb.at[s_v], g_v)
        pltpu.sync_copy(edge_emb.at[pl.ds(c * CHUNK, CHUNK)], e_v)
        pltpu.sync_copy(g_v, acc1.at[d_v], add=True)
        pltpu.sync_copy(e_v, acc2.at[d_v], add=True)

    def body(k, carry):
        do_chunk(w + k * N_WORKERS)
        return carry

    lax.fori_loop(0, FULL_ROUNDS, body, 0, unroll=False)

    @pl.when(w < TAIL)
    def _tail():
        do_chunk(FULL_ROUNDS * N_WORKERS + w)
    plsc.subcore_barrier()

    pltpu.sync_copy(
        acc1.at[pl.ds(sid * ZROWS, ZROWS)],
        out1.at[pl.ds(cid * N_NODES + sid * ZROWS, ZROWS)])
    pltpu.sync_copy(
        acc2.at[pl.ds(sid * ZROWS, ZROWS)],
        out2.at[pl.ds(cid * N_NODES + sid * ZROWS, ZROWS)])


@jax.jit
def sc_feat(node_emb, edge_emb, src, dst, zeros):
    return pl.kernel(
        _sc_feat_body,
        out_type=(
            jax.ShapeDtypeStruct((SC_CORES * N_NODES, D_NODE), jnp.float32),
            jax.ShapeDtypeStruct((SC_CORES * N_NODES, D_EOUT), jnp.float32),
        ),
        mesh=_mesh(),
        compiler_params=pltpu.CompilerParams(use_tc_tiling_on_sc=False),
        scratch_types=[
            pltpu.VMEM((CHUNK,), jnp.int32),
            pltpu.VMEM((CHUNK,), jnp.int32),
            pltpu.VMEM((CHUNK, D_NODE), jnp.float32),
            pltpu.VMEM((CHUNK, D_EOUT), jnp.float32),
            pltpu.VMEM_SHARED((N_NODES, D_NODE), jnp.float32),
            pltpu.VMEM_SHARED((N_NODES, D_EOUT), jnp.float32),
        ],
    )(node_emb, edge_emb, src, dst, zeros)


def _tc_node_proj_body(x_ref, w_ref, b_ref, o_ref):
    o_ref[...] = jax.nn.relu(_dot(x_ref[...], w_ref[...]) + b_ref[...])


@jax.jit
def tc_node_proj(x, w, b):
    return pl.pallas_call(
        _tc_node_proj_body,
        out_shape=jax.ShapeDtypeStruct((N_NODES, D_NODE), jnp.float32),
    )(x, w, b)


EB_PREP = 8000


def _tc_edge_prep_body(ea_ref, w1_ref, b1_ref, wp_ref, bp_ref, z_ref, ee_ref):
    ea = ea_ref[...]
    z_ref[...] = jax.nn.relu(_dot(ea, w1_ref[...]) + b1_ref[...])
    ee_ref[...] = jax.nn.relu(_dot(ea, wp_ref[...]) + bp_ref[...])


@jax.jit
def tc_edge_prep(ea, w1, b1, wp, bp):
    nb = N_EDGES // EB_PREP
    return pl.pallas_call(
        _tc_edge_prep_body,
        grid=(nb,),
        in_specs=[
            pl.BlockSpec((EB_PREP, D_BOND), lambda i: (i, 0)),
            pl.BlockSpec((D_BOND, D_EH), lambda i: (0, 0)),
            pl.BlockSpec((1, D_EH), lambda i: (0, 0)),
            pl.BlockSpec((D_BOND, D_EOUT), lambda i: (0, 0)),
            pl.BlockSpec((1, D_EOUT), lambda i: (0, 0)),
        ],
        out_specs=[
            pl.BlockSpec((EB_PREP, D_EH), lambda i: (i, 0)),
            pl.BlockSpec((EB_PREP, D_EOUT), lambda i: (i, 0)),
        ],
        out_shape=[
            jax.ShapeDtypeStruct((N_EDGES, D_EH), jnp.float32),
            jax.ShapeDtypeStruct((N_EDGES, D_EOUT), jnp.float32),
        ],
    )(ea, w1, b1, wp, bp)


EB_MSG = 2000


def _tc_msg_body(z_ref, g_ref, w2_ref, b2_ref, msg_ref):
    ew = _dot(z_ref[...], w2_ref[...]) + b2_ref[...]
    g = g_ref[...]
    acc = jnp.zeros((EB_MSG, D_NODE), jnp.float32)
    for i in range(D_NODE):
        acc = acc + g[:, i:i + 1] * ew[:, i * D_NODE:(i + 1) * D_NODE]
    msg_ref[...] = acc


@jax.jit
def tc_msg(z, g, w2, b2):
    nb = N_EDGES // EB_MSG
    return pl.pallas_call(
        _tc_msg_body,
        grid=(nb,),
        in_specs=[
            pl.BlockSpec((EB_MSG, D_EH), lambda i: (i, 0)),
            pl.BlockSpec((EB_MSG, D_NODE), lambda i: (i, 0)),
            pl.BlockSpec((D_EH, D_NODE * D_NODE), lambda i: (0, 0)),
            pl.BlockSpec((1, D_NODE * D_NODE), lambda i: (0, 0)),
        ],
        out_specs=pl.BlockSpec((EB_MSG, D_NODE), lambda i: (i, 0)),
        out_shape=jax.ShapeDtypeStruct((N_EDGES, D_NODE), jnp.float32),
    )(z, g, w2, b2)


def _tc_gru_body(aggp_ref, bc_ref, node_ref, hid_ref, wih_ref, whh_ref,
                 bih_ref, bhh_ref, node_o_ref, hid_o_ref):
    agg = aggp_ref[0] + aggp_ref[1] + bc_ref[...]
    m = jax.nn.relu(agg)
    hidden = hid_ref[...]
    gi = _dot(m, wih_ref[...]) + bih_ref[...]
    gh = _dot(hidden, whh_ref[...]) + bhh_ref[...]
    i_r = gi[:, 0 * D_NODE:1 * D_NODE]
    i_z = gi[:, 1 * D_NODE:2 * D_NODE]
    i_n = gi[:, 2 * D_NODE:3 * D_NODE]
    h_r = gh[:, 0 * D_NODE:1 * D_NODE]
    h_z = gh[:, 1 * D_NODE:2 * D_NODE]
    h_n = gh[:, 2 * D_NODE:3 * D_NODE]
    r = jax.nn.sigmoid(i_r + h_r)
    zg = jax.nn.sigmoid(i_z + h_z)
    n_ = jnp.tanh(i_n + r * h_n)
    new_h = (1.0 - zg) * n_ + zg * hidden
    node_o_ref[...] = new_h + node_ref[...]
    hid_o_ref[...] = new_h


NB_GRU = 2000


@jax.jit
def tc_gru(aggp, bc, node, hidden, wih, whh, bih, bhh):
    aggp = aggp.reshape(SC_CORES, N_NODES, D_NODE)
    nb = N_NODES // NB_GRU
    return pl.pallas_call(
        _tc_gru_body,
        grid=(nb,),
        in_specs=[
            pl.BlockSpec((SC_CORES, NB_GRU, D_NODE), lambda i: (0, i, 0)),
            pl.BlockSpec((1, D_NODE), lambda i: (0, 0)),
            pl.BlockSpec((NB_GRU, D_NODE), lambda i: (i, 0)),
            pl.BlockSpec((NB_GRU, D_NODE), lambda i: (i, 0)),
            pl.BlockSpec((D_NODE, 3 * D_NODE), lambda i: (0, 0)),
            pl.BlockSpec((D_NODE, 3 * D_NODE), lambda i: (0, 0)),
            pl.BlockSpec((1, 3 * D_NODE), lambda i: (0, 0)),
            pl.BlockSpec((1, 3 * D_NODE), lambda i: (0, 0)),
        ],
        out_specs=[
            pl.BlockSpec((NB_GRU, D_NODE), lambda i: (i, 0)),
            pl.BlockSpec((NB_GRU, D_NODE), lambda i: (i, 0)),
        ],
        out_shape=[
            jax.ShapeDtypeStruct((N_NODES, D_NODE), jnp.float32),
            jax.ShapeDtypeStruct((N_NODES, D_NODE), jnp.float32),
        ],
    )(aggp, bc, node, hidden, wih, whh, bih, bhh)


NB_S2S = 2000
NBLK_S2S = N_NODES // NB_S2S


def _tlhs_dot(a, b):
    return lax.dot_general(a, b, (((0,), (0,)), ((), ())),
                           precision=_HIGH, preferred_element_type=jnp.float32)


def _tc_s2s_body(f1p_ref, f2p_ref, gid_ref,
                 wih0_ref, whh0_ref, b0_ref,
                 wih1_ref, whh1_ref, b1_ref,
                 wih2_ref, whh2_ref, b2_ref,
                 wf1_ref, bf1_ref, wf2_ref, bf2_ref,
                 wf3a_ref, wf3b_ref, bf3a_ref, bf3b_ref,
                 p0_ref, p1_ref,
                 q_s, rt_s, m_s, s_s, acc_s,
                 h0_s, c0_s, h1_s, c1_s, h2_s, c2_s):
    it = pl.program_id(0)
    blk = pl.program_id(1)

    @pl.when(jnp.logical_and(it == 0, blk == 0))
    def _init():
        q_s[...] = jnp.zeros_like(q_s)
        rt_s[...] = jnp.zeros_like(rt_s)
        h0_s[...] = jnp.zeros_like(h0_s)
        c0_s[...] = jnp.zeros_like(c0_s)
        h1_s[...] = jnp.zeros_like(h1_s)
        c1_s[...] = jnp.zeros_like(c1_s)
        h2_s[...] = jnp.zeros_like(h2_s)
        c2_s[...] = jnp.zeros_like(c2_s)

    @pl.when(blk == 0)
    def _lstm():
        q_prev = q_s[...]
        rt_prev = rt_s[...]
        hs = [h0_s, h1_s, h2_s]
        cs = [c0_s, c1_s, c2_s]
        wih = [wih0_ref, wih1_ref, wih2_ref]
        whh = [whh0_ref, whh1_ref, whh2_ref]
        bb = [b0_ref, b1_ref, b2_ref]
        inp = None
        for l in range(LAYERS):
            if l == 0:
                gates = (_dot(q_prev, wih0_ref[0:DIM, :])
                         + _tlhs_dot(rt_prev, wih0_ref[DIM:2 * DIM, :]))
            else:
                gates = _dot(inp, wih[l][...])
            gates = gates + _dot(hs[l][...], whh[l][...]) + bb[l][...]
            ig = gates[:, 0 * DIM:1 * DIM]
            fg = gates[:, 1 * DIM:2 * DIM]
            gg = gates[:, 2 * DIM:3 * DIM]
            og = gates[:, 3 * DIM:4 * DIM]
            c_new = (jax.nn.sigmoid(fg) * cs[l][...]
                     + jax.nn.sigmoid(ig) * jnp.tanh(gg))
            h_new = jax.nn.sigmoid(og) * jnp.tanh(c_new)
            cs[l][...] = c_new
            hs[l][...] = h_new
            inp = h_new
        q_s[...] = inp
        m_s[...] = jnp.full_like(m_s, -1e30)
        s_s[...] = jnp.zeros_like(s_s)
        acc_s[...] = jnp.zeros_like(acc_s)

    featb = jnp.concatenate(
        [f1p_ref[0] + f1p_ref[1], f2p_ref[0] + f2p_ref[1]], axis=1)
    gid = gid_ref[...]
    iota_g = lax.broadcasted_iota(jnp.int32, (NB_S2S, N_GRAPHS), 1)
    mask = jnp.where(gid == iota_g, 1.0, 0.0)
    qg = _dot(mask, q_s[...])
    e = jnp.sum(featb * qg, axis=1, keepdims=True)
    col = jnp.where(mask > 0.0, e, -1e30)
    bmax = jnp.max(col, axis=0, keepdims=True)
    m_old = m_s[...]
    m_new = jnp.maximum(m_old, bmax)
    scale = jnp.exp(m_old - m_new)
    emaxg = jnp.sum(mask * m_new, axis=1, keepdims=True)
    exn = jnp.exp(e - emaxg)
    maskex = mask * exn
    s_s[...] = s_s[...] * scale + jnp.sum(maskex, axis=0, keepdims=True)
    acc_s[...] = acc_s[...] * scale + _tlhs_dot(featb, maskex)
    m_s[...] = m_new

    @pl.when(blk == NBLK_S2S - 1)
    def _finish():
        rt_s[...] = acc_s[...] / (s_s[...] + 1e-12)

    @pl.when(jnp.logical_and(it == ITERS - 1, blk == NBLK_S2S - 1))
    def _mlp():
        rt = acc_s[...] / (s_s[...] + 1e-12)
        hdn = jax.nn.relu(_dot(q_s[...], wf1_ref[0:DIM, :])
                          + _tlhs_dot(rt, wf1_ref[DIM:2 * DIM, :])
                          + bf1_ref[...])
        hdn = jax.nn.relu(_dot(hdn, wf2_ref[...]) + bf2_ref[...])
        l0 = _dot(hdn, wf3a_ref[...]) + bf3a_ref[...]
        l1 = _dot(hdn, wf3b_ref[...]) + bf3b_ref[...]
        mm = jnp.maximum(l0, l1)
        e0 = jnp.exp(l0 - mm)
        e1 = jnp.exp(l1 - mm)
        tot = e0 + e1
        p0_ref[...] = e0 / tot
        p1_ref[...] = e1 / tot


@jax.jit
def tc_tail(f1p, f2p, gid_col, args):
    f1p = f1p.reshape(SC_CORES, N_NODES, D_NODE)
    f2p = f2p.reshape(SC_CORES, N_NODES, D_EOUT)
    full = lambda shape: pl.BlockSpec(shape, lambda it, b: tuple(0 for _ in shape))
    return pl.pallas_call(
        _tc_s2s_body,
        grid=(ITERS, NBLK_S2S),
        in_specs=[
            pl.BlockSpec((SC_CORES, NB_S2S, D_NODE), lambda it, b: (0, b, 0)),
            pl.BlockSpec((SC_CORES, NB_S2S, D_EOUT), lambda it, b: (0, b, 0)),
            pl.BlockSpec((NB_S2S, 1), lambda it, b: (b, 0)),
            full((2 * DIM, 4 * DIM)), full((DIM, 4 * DIM)), full((1, 4 * DIM)),
            full((DIM, 4 * DIM)), full((DIM, 4 * DIM)), full((1, 4 * DIM)),
            full((DIM, 4 * DIM)), full((DIM, 4 * DIM)), full((1, 4 * DIM)),
            full((2 * DIM, 300)), full((1, 300)),
            full((300, 256)), full((1, 256)),
            full((256, N_TASKS)), full((256, N_TASKS)),
            full((1, N_TASKS)), full((1, N_TASKS)),
        ],
        out_specs=[
            pl.BlockSpec((N_GRAPHS, N_TASKS), lambda it, b: (0, 0)),
            pl.BlockSpec((N_GRAPHS, N_TASKS), lambda it, b: (0, 0)),
        ],
        out_shape=[
            jax.ShapeDtypeStruct((N_GRAPHS, N_TASKS), jnp.float32),
            jax.ShapeDtypeStruct((N_GRAPHS, N_TASKS), jnp.float32),
        ],
        scratch_shapes=[
            pltpu.VMEM((N_GRAPHS, DIM), jnp.float32),
            pltpu.VMEM((DIM, N_GRAPHS), jnp.float32),
            pltpu.VMEM((1, N_GRAPHS), jnp.float32),
            pltpu.VMEM((1, N_GRAPHS), jnp.float32),
            pltpu.VMEM((DIM, N_GRAPHS), jnp.float32),
            pltpu.VMEM((N_GRAPHS, DIM), jnp.float32),
            pltpu.VMEM((N_GRAPHS, DIM), jnp.float32),
            pltpu.VMEM((N_GRAPHS, DIM), jnp.float32),
            pltpu.VMEM((N_GRAPHS, DIM), jnp.float32),
            pltpu.VMEM((N_GRAPHS, DIM), jnp.float32),
            pltpu.VMEM((N_GRAPHS, DIM), jnp.float32),
        ],
        compiler_params=pltpu.CompilerParams(
            dimension_semantics=("arbitrary", "arbitrary")),
    )(f1p, f2p, gid_col, *args)


def kernel(x, edge_index, edge_attr, graph_ids, params):
    p = params
    src = edge_index[0]
    dst = edge_index[1]
    zeros = jnp.zeros((ZROWS, D_NODE), jnp.float32)

    h = tc_node_proj(x, p['W_proj'], p['b_proj'].reshape(1, -1))
    z, ee = tc_edge_prep(edge_attr, p['We1'], p['be1'].reshape(1, -1),
                         p['Wpe'], p['bpe'].reshape(1, -1))

    node = h
    hidden = h
    for _ in range(STEPS):
        g = sc_gather(node, src)
        msg = tc_msg(z, g, p['We2'], p['be2'].reshape(1, -1))
        aggp = sc_scatter_add(msg, dst, zeros)
        node, hidden = tc_gru(aggp, p['b_conv'].reshape(1, -1), node, hidden,
                              p['Wih_gru'], p['Whh_gru'],
                              p['bih_gru'].reshape(1, -1),
                              p['bhh_gru'].reshape(1, -1))

    f1p, f2p = sc_feat(node, ee, src, dst, zeros)

    wf3 = p['Wf3'].reshape(-1, N_TASKS, N_CLASSES)
    bf3 = p['bf3'].reshape(N_TASKS, N_CLASSES)
    tail_args = (
        p['lstm0_Wih'], p['lstm0_Whh'],
        (p['lstm0_bih'] + p['lstm0_bhh']).reshape(1, -1),
        p['lstm1_Wih'], p['lstm1_Whh'],
        (p['lstm1_bih'] + p['lstm1_bhh']).reshape(1, -1),
        p['lstm2_Wih'], p['lstm2_Whh'],
        (p['lstm2_bih'] + p['lstm2_bhh']).reshape(1, -1),
        p['Wf1'], p['bf1'].reshape(1, -1),
        p['Wf2'], p['bf2'].reshape(1, -1),
        wf3[:, :, 0], wf3[:, :, 1],
        bf3[:, 0].reshape(1, -1), bf3[:, 1].reshape(1, -1),
    )
    p0, p1 = tc_tail(f1p, f2p, graph_ids.reshape(-1, 1), tail_args)
    return jnp.stack([p0, p1], axis=-1)


# speedup vs baseline: 2.9416x; 2.6554x over previous
import functools

import jax
import jax.numpy as jnp
from jax import lax
from jax.experimental import pallas as pl
from jax.experimental.pallas import tpu as pltpu
from jax.experimental.pallas import tpu_sc as plsc

N_NODES = 10000
N_EDGES = 160000
N_GRAPHS = 256
D_ATOM = 30
D_BOND = 11
D_NODE = 32
D_EH = 128
D_EOUT = 32
DIM = D_NODE + D_EOUT
STEPS = 3
ITERS = 6
LAYERS = 3
N_TASKS = 12
N_CLASSES = 2

SC_CORES = 2
SC_SUBCORES = 16
N_WORKERS = SC_CORES * SC_SUBCORES
CHUNK = 128
N_CHUNKS = N_EDGES // CHUNK
FULL_ROUNDS = N_CHUNKS // N_WORKERS
TAIL = N_CHUNKS - FULL_ROUNDS * N_WORKERS
ZROWS = N_NODES // SC_SUBCORES

@functools.cache
def _mesh():
    return plsc.VectorSubcoreMesh(
        core_axis_name="c", subcore_axis_name="s",
        num_cores=SC_CORES, num_subcores=SC_SUBCORES)

def _dot(a, b):
    return jnp.dot(a, b, preferred_element_type=jnp.float32)


def _doth(a, b):
    return jnp.dot(a, b, preferred_element_type=jnp.float32)


def _wid():
    return lax.axis_index("s") * SC_CORES + lax.axis_index("c")


def _sc_gather_body(table, idx, out, idx_v, rows_v):
    w = _wid()

    def do_chunk(c):
        pltpu.sync_copy(idx.at[pl.ds(c * CHUNK, CHUNK)], idx_v)
        pltpu.sync_copy(table.at[idx_v], rows_v)
        pltpu.sync_copy(rows_v, out.at[pl.ds(c * CHUNK, CHUNK)])

    def body(k, carry):
        do_chunk(w + k * N_WORKERS)
        return carry

    lax.fori_loop(0, FULL_ROUNDS, body, 0, unroll=False)

    @pl.when(w < TAIL)
    def _tail():
        do_chunk(FULL_ROUNDS * N_WORKERS + w)


@jax.jit
def sc_gather(table, idx):
    return pl.kernel(
        _sc_gather_body,
        out_type=jax.ShapeDtypeStruct((N_EDGES, D_NODE), jnp.float32),
        mesh=_mesh(),
        compiler_params=pltpu.CompilerParams(use_tc_tiling_on_sc=False),
        scratch_types=[
            pltpu.VMEM((CHUNK,), jnp.int32),
            pltpu.VMEM((CHUNK, D_NODE), jnp.float32),
        ],
    )(table, idx)


def _sc_scatter_body(vals, idx, zeros, out, idx_v, v_v, acc):
    w = _wid()
    sid = lax.axis_index("s")
    cid = lax.axis_index("c")

    pltpu.sync_copy(zeros, acc.at[pl.ds(sid * ZROWS, ZROWS)])
    plsc.subcore_barrier()

    def do_chunk(c):
        pltpu.sync_copy(idx.at[pl.ds(c * CHUNK, CHUNK)], idx_v)
        pltpu.sync_copy(vals.at[pl.ds(c * CHUNK, CHUNK)], v_v)
        pltpu.sync_copy(v_v, acc.at[idx_v], add=True)

    def body(k, carry):
        do_chunk(w + k * N_WORKERS)
        return carry

    lax.fori_loop(0, FULL_ROUNDS, body, 0, unroll=False)

    @pl.when(w < TAIL)
    def _tail():
        do_chunk(FULL_ROUNDS * N_WORKERS + w)
    plsc.subcore_barrier()

    pltpu.sync_copy(
        acc.at[pl.ds(sid * ZROWS, ZROWS)],
        out.at[pl.ds(cid * N_NODES + sid * ZROWS, ZROWS)])


@jax.jit
def sc_scatter_add(vals, idx, zeros):
    return pl.kernel(
        _sc_scatter_body,
        out_type=jax.ShapeDtypeStruct((SC_CORES * N_NODES, D_NODE), jnp.float32),
        mesh=_mesh(),
        compiler_params=pltpu.CompilerParams(use_tc_tiling_on_sc=False),
        scratch_types=[
            pltpu.VMEM((CHUNK,), jnp.int32),
            pltpu.VMEM((CHUNK, D_NODE), jnp.float32),
            pltpu.VMEM_SHARED((N_NODES, D_NODE), jnp.float32),
        ],
    )(vals, idx, zeros)


def _sc_feat_body(node_emb, edge_emb, src, dst, zeros, out1, out2,
                  s_v, d_v, g_v, e_v, acc1, acc2):
    w = _wid()
    sid = lax.axis_index("s")
    cid = lax.axis_index("c")

    pltpu.sync_copy(zeros, acc1.at[pl.ds(sid * ZROWS, ZROWS)])
    pltpu.sync_copy(zeros, acc2.at[pl.ds(sid * ZROWS, ZROWS)])
    plsc.subcore_barrier()

    def do_chunk(c):
        pltpu.sync_copy(src.at[pl.ds(c * CHUNK, CHUNK)], s_v)
        pltpu.sync_copy(dst.at[pl.ds(c * CHUNK, CHUNK)], d_v)
        pltpu.sync_copy(node_emb.at[s_v], g_v)
        pltpu.sync_copy(edge_emb.at[pl.ds(c * CHUNK, CHUNK)], e_v)
        pltpu.sync_copy(g_v, acc1.at[d_v], add=True)
        pltpu.sync_copy(e_v, acc2.at[d_v], add=True)

    def body(k, carry):
        do_chunk(w + k * N_WORKERS)
        return carry

    lax.fori_loop(0, FULL_ROUNDS, body, 0, unroll=False)

    @pl.when(w < TAIL)
    def _tail():
        do_chunk(FULL_ROUNDS * N_WORKERS + w)
    plsc.subcore_barrier()

    pltpu.sync_copy(
        acc1.at[pl.ds(sid * ZROWS, ZROWS)],
        out1.at[pl.ds(cid * N_NODES + sid * ZROWS, ZROWS)])
    pltpu.sync_copy(
        acc2.at[pl.ds(sid * ZROWS, ZROWS)],
        out2.at[pl.ds(cid * N_NODES + sid * ZROWS, ZROWS)])


@jax.jit
def sc_feat(node_emb, edge_emb, src, dst, zeros):
    return pl.kernel(
        _sc_feat_body,
        out_type=(
            jax.ShapeDtypeStruct((SC_CORES * N_NODES, D_NODE), jnp.float32),
            jax.ShapeDtypeStruct((SC_CORES * N_NODES, D_EOUT), jnp.float32),
        ),
        mesh=_mesh(),
        compiler_params=pltpu.CompilerParams(use_tc_tiling_on_sc=False),
        scratch_types=[
            pltpu.VMEM((CHUNK,), jnp.int32),
            pltpu.VMEM((CHUNK,), jnp.int32),
            pltpu.VMEM((CHUNK, D_NODE), jnp.float32),
            pltpu.VMEM((CHUNK, D_EOUT), jnp.float32),
            pltpu.VMEM_SHARED((N_NODES, D_NODE), jnp.float32),
            pltpu.VMEM_SHARED((N_NODES, D_EOUT), jnp.float32),
        ],
    )(node_emb, edge_emb, src, dst, zeros)


def _tc_node_proj_body(x_ref, w_ref, b_ref, o_ref):
    o_ref[...] = jax.nn.relu(_dot(x_ref[...], w_ref[...]) + b_ref[...])


@jax.jit
def tc_node_proj(x, w, b):
    return pl.pallas_call(
        _tc_node_proj_body,
        out_shape=jax.ShapeDtypeStruct((N_NODES, D_NODE), jnp.float32),
    )(x, w, b)


EB_PREP = 8000


def _tc_edge_prep_body(ea_ref, w1_ref, b1_ref, wp_ref, bp_ref, z_ref, ee_ref):
    ea = ea_ref[...]
    z_ref[...] = jax.nn.relu(_dot(ea, w1_ref[...]) + b1_ref[...])
    ee_ref[...] = jax.nn.relu(_dot(ea, wp_ref[...]) + bp_ref[...])


@jax.jit
def tc_edge_prep(ea, w1, b1, wp, bp):
    nb = N_EDGES // EB_PREP
    return pl.pallas_call(
        _tc_edge_prep_body,
        grid=(nb,),
        in_specs=[
            pl.BlockSpec((EB_PREP, D_BOND), lambda i: (i, 0)),
            pl.BlockSpec((D_BOND, D_EH), lambda i: (0, 0)),
            pl.BlockSpec((1, D_EH), lambda i: (0, 0)),
            pl.BlockSpec((D_BOND, D_EOUT), lambda i: (0, 0)),
            pl.BlockSpec((1, D_EOUT), lambda i: (0, 0)),
        ],
        out_specs=[
            pl.BlockSpec((EB_PREP, D_EH), lambda i: (i, 0)),
            pl.BlockSpec((EB_PREP, D_EOUT), lambda i: (i, 0)),
        ],
        out_shape=[
            jax.ShapeDtypeStruct((N_EDGES, D_EH), jnp.float32),
            jax.ShapeDtypeStruct((N_EDGES, D_EOUT), jnp.float32),
        ],
    )(ea, w1, b1, wp, bp)


EB_MSG = 2000


def _tc_msg_body(z_ref, g_ref, w2_ref, b2_ref, r_ref, s_ref, msg_ref):
    ew = _doth(z_ref[...], w2_ref[...]) + b2_ref[...]
    gbig = _doth(g_ref[...], r_ref[...])
    msg_ref[...] = _doth(ew * gbig, s_ref[...])


@jax.jit
def tc_msg(z, g, w2, b2, rmat, smat):
    nb = N_EDGES // EB_MSG
    return pl.pallas_call(
        _tc_msg_body,
        grid=(nb,),
        in_specs=[
            pl.BlockSpec((EB_MSG, D_EH), lambda i: (i, 0)),
            pl.BlockSpec((EB_MSG, D_NODE), lambda i: (i, 0)),
            pl.BlockSpec((D_EH, D_NODE * D_NODE), lambda i: (0, 0)),
            pl.BlockSpec((1, D_NODE * D_NODE), lambda i: (0, 0)),
            pl.BlockSpec((D_NODE, D_NODE * D_NODE), lambda i: (0, 0)),
            pl.BlockSpec((D_NODE * D_NODE, D_NODE), lambda i: (0, 0)),
        ],
        out_specs=pl.BlockSpec((EB_MSG, D_NODE), lambda i: (i, 0)),
        out_shape=jax.ShapeDtypeStruct((N_EDGES, D_NODE), jnp.float32),
    )(z, g, w2, b2, rmat, smat)


def _tc_gru_body(aggp_ref, bc_ref, node_ref, hid_ref, wih_ref, whh_ref,
                 bih_ref, bhh_ref, node_o_ref, hid_o_ref):
    agg = aggp_ref[0] + aggp_ref[1] + bc_ref[...]
    m = jax.nn.relu(agg)
    hidden = hid_ref[...]
    gi = _dot(m, wih_ref[...]) + bih_ref[...]
    gh = _dot(hidden, whh_ref[...]) + bhh_ref[...]
    i_r = gi[:, 0 * D_NODE:1 * D_NODE]
    i_z = gi[:, 1 * D_NODE:2 * D_NODE]
    i_n = gi[:, 2 * D_NODE:3 * D_NODE]
    h_r = gh[:, 0 * D_NODE:1 * D_NODE]
    h_z = gh[:, 1 * D_NODE:2 * D_NODE]
    h_n = gh[:, 2 * D_NODE:3 * D_NODE]
    r = jax.nn.sigmoid(i_r + h_r)
    zg = jax.nn.sigmoid(i_z + h_z)
    n_ = jnp.tanh(i_n + r * h_n)
    new_h = (1.0 - zg) * n_ + zg * hidden
    node_o_ref[...] = new_h + node_ref[...]
    hid_o_ref[...] = new_h


NB_GRU = 2000


@jax.jit
def tc_gru(aggp, bc, node, hidden, wih, whh, bih, bhh):
    aggp = aggp.reshape(SC_CORES, N_NODES, D_NODE)
    nb = N_NODES // NB_GRU
    return pl.pallas_call(
        _tc_gru_body,
        grid=(nb,),
        in_specs=[
            pl.BlockSpec((SC_CORES, NB_GRU, D_NODE), lambda i: (0, i, 0)),
            pl.BlockSpec((1, D_NODE), lambda i: (0, 0)),
            pl.BlockSpec((NB_GRU, D_NODE), lambda i: (i, 0)),
            pl.BlockSpec((NB_GRU, D_NODE), lambda i: (i, 0)),
            pl.BlockSpec((D_NODE, 3 * D_NODE), lambda i: (0, 0)),
            pl.BlockSpec((D_NODE, 3 * D_NODE), lambda i: (0, 0)),
            pl.BlockSpec((1, 3 * D_NODE), lambda i: (0, 0)),
            pl.BlockSpec((1, 3 * D_NODE), lambda i: (0, 0)),
        ],
        out_specs=[
            pl.BlockSpec((NB_GRU, D_NODE), lambda i: (i, 0)),
            pl.BlockSpec((NB_GRU, D_NODE), lambda i: (i, 0)),
        ],
        out_shape=[
            jax.ShapeDtypeStruct((N_NODES, D_NODE), jnp.float32),
            jax.ShapeDtypeStruct((N_NODES, D_NODE), jnp.float32),
        ],
    )(aggp, bc, node, hidden, wih, whh, bih, bhh)


NB_S2S = 2000
NBLK_S2S = N_NODES // NB_S2S


def _tlhs_dot(a, b):
    return lax.dot_general(a, b, (((0,), (0,)), ((), ())),
                           preferred_element_type=jnp.float32)


def _tc_s2s_body(f1p_ref, f2p_ref, gid_ref,
                 wih0_ref, whh0_ref, b0_ref,
                 wih1_ref, whh1_ref, b1_ref,
                 wih2_ref, whh2_ref, b2_ref,
                 wf1_ref, bf1_ref, wf2_ref, bf2_ref,
                 wf3a_ref, wf3b_ref, bf3a_ref, bf3b_ref,
                 p0_ref, p1_ref,
                 q_s, rt_s, m_s, s_s, acc_s,
                 h0_s, c0_s, h1_s, c1_s, h2_s, c2_s):
    it = pl.program_id(0)
    blk = pl.program_id(1)

    @pl.when(jnp.logical_and(it == 0, blk == 0))
    def _init():
        q_s[...] = jnp.zeros_like(q_s)
        rt_s[...] = jnp.zeros_like(rt_s)
        h0_s[...] = jnp.zeros_like(h0_s)
        c0_s[...] = jnp.zeros_like(c0_s)
        h1_s[...] = jnp.zeros_like(h1_s)
        c1_s[...] = jnp.zeros_like(c1_s)
        h2_s[...] = jnp.zeros_like(h2_s)
        c2_s[...] = jnp.zeros_like(c2_s)

    @pl.when(blk == 0)
    def _lstm():
        q_prev = q_s[...]
        rt_prev = rt_s[...]
        hs = [h0_s, h1_s, h2_s]
        cs = [c0_s, c1_s, c2_s]
        wih = [wih0_ref, wih1_ref, wih2_ref]
        whh = [whh0_ref, whh1_ref, whh2_ref]
        bb = [b0_ref, b1_ref, b2_ref]
        inp = None
        for l in range(LAYERS):
            if l == 0:
                gates = (_dot(q_prev, wih0_ref[0:DIM, :])
                         + _tlhs_dot(rt_prev, wih0_ref[DIM:2 * DIM, :]))
            else:
                gates = _dot(inp, wih[l][...])
            gates = gates + _dot(hs[l][...], whh[l][...]) + bb[l][...]
            ig = gates[:, 0 * DIM:1 * DIM]
            fg = gates[:, 1 * DIM:2 * DIM]
            gg = gates[:, 2 * DIM:3 * DIM]
            og = gates[:, 3 * DIM:4 * DIM]
            c_new = (jax.nn.sigmoid(fg) * cs[l][...]
                     + jax.nn.sigmoid(ig) * jnp.tanh(gg))
            h_new = jax.nn.sigmoid(og) * jnp.tanh(c_new)
            cs[l][...] = c_new
            hs[l][...] = h_new
            inp = h_new
        q_s[...] = inp
        m_s[...] = jnp.full_like(m_s, -1e30)
        s_s[...] = jnp.zeros_like(s_s)
        acc_s[...] = jnp.zeros_like(acc_s)

    featb = jnp.concatenate(
        [f1p_ref[0] + f1p_ref[1], f2p_ref[0] + f2p_ref[1]], axis=1)
    gid = gid_ref[...]
    iota_g = lax.broadcasted_iota(jnp.int32, (NB_S2S, N_GRAPHS), 1)
    mask = jnp.where(gid == iota_g, 1.0, 0.0)
    qg = _dot(mask, q_s[...])
    e = jnp.sum(featb * qg, axis=1, keepdims=True)
    col = jnp.where(mask > 0.0, e, -1e30)
    bmax = jnp.max(col, axis=0, keepdims=True)
    m_old = m_s[...]
    m_new = jnp.maximum(m_old, bmax)
    scale = jnp.exp(m_old - m_new)
    emaxg = jnp.sum(mask * m_new, axis=1, keepdims=True)
    exn = jnp.exp(e - emaxg)
    maskex = mask * exn
    s_s[...] = s_s[...] * scale + jnp.sum(maskex, axis=0, keepdims=True)
    acc_s[...] = acc_s[...] * scale + _tlhs_dot(featb, maskex)
    m_s[...] = m_new

    @pl.when(blk == NBLK_S2S - 1)
    def _finish():
        rt_s[...] = acc_s[...] / (s_s[...] + 1e-12)

    @pl.when(jnp.logical_and(it == ITERS - 1, blk == NBLK_S2S - 1))
    def _mlp():
        rt = acc_s[...] / (s_s[...] + 1e-12)
        hdn = jax.nn.relu(_dot(q_s[...], wf1_ref[0:DIM, :])
                          + _tlhs_dot(rt, wf1_ref[DIM:2 * DIM, :])
                          + bf1_ref[...])
        hdn = jax.nn.relu(_dot(hdn, wf2_ref[...]) + bf2_ref[...])
        l0 = _dot(hdn, wf3a_ref[...]) + bf3a_ref[...]
        l1 = _dot(hdn, wf3b_ref[...]) + bf3b_ref[...]
        mm = jnp.maximum(l0, l1)
        e0 = jnp.exp(l0 - mm)
        e1 = jnp.exp(l1 - mm)
        tot = e0 + e1
        p0_ref[...] = e0 / tot
        p1_ref[...] = e1 / tot


@jax.jit
def tc_tail(f1p, f2p, gid_col, args):
    f1p = f1p.reshape(SC_CORES, N_NODES, D_NODE)
    f2p = f2p.reshape(SC_CORES, N_NODES, D_EOUT)
    full = lambda shape: pl.BlockSpec(shape, lambda it, b: tuple(0 for _ in shape))
    return pl.pallas_call(
        _tc_s2s_body,
        grid=(ITERS, NBLK_S2S),
        in_specs=[
            pl.BlockSpec((SC_CORES, NB_S2S, D_NODE), lambda it, b: (0, b, 0)),
            pl.BlockSpec((SC_CORES, NB_S2S, D_EOUT), lambda it, b: (0, b, 0)),
            pl.BlockSpec((NB_S2S, 1), lambda it, b: (b, 0)),
            full((2 * DIM, 4 * DIM)), full((DIM, 4 * DIM)), full((1, 4 * DIM)),
            full((DIM, 4 * DIM)), full((DIM, 4 * DIM)), full((1, 4 * DIM)),
            full((DIM, 4 * DIM)), full((DIM, 4 * DIM)), full((1, 4 * DIM)),
            full((2 * DIM, 300)), full((1, 300)),
            full((300, 256)), full((1, 256)),
            full((256, N_TASKS)), full((256, N_TASKS)),
            full((1, N_TASKS)), full((1, N_TASKS)),
        ],
        out_specs=[
            pl.BlockSpec((N_GRAPHS, N_TASKS), lambda it, b: (0, 0)),
            pl.BlockSpec((N_GRAPHS, N_TASKS), lambda it, b: (0, 0)),
        ],
        out_shape=[
            jax.ShapeDtypeStruct((N_GRAPHS, N_TASKS), jnp.float32),
            jax.ShapeDtypeStruct((N_GRAPHS, N_TASKS), jnp.float32),
        ],
        scratch_shapes=[
            pltpu.VMEM((N_GRAPHS, DIM), jnp.float32),
            pltpu.VMEM((DIM, N_GRAPHS), jnp.float32),
            pltpu.VMEM((1, N_GRAPHS), jnp.float32),
            pltpu.VMEM((1, N_GRAPHS), jnp.float32),
            pltpu.VMEM((DIM, N_GRAPHS), jnp.float32),
            pltpu.VMEM((N_GRAPHS, DIM), jnp.float32),
            pltpu.VMEM((N_GRAPHS, DIM), jnp.float32),
            pltpu.VMEM((N_GRAPHS, DIM), jnp.float32),
            pltpu.VMEM((N_GRAPHS, DIM), jnp.float32),
            pltpu.VMEM((N_GRAPHS, DIM), jnp.float32),
            pltpu.VMEM((N_GRAPHS, DIM), jnp.float32),
        ],
        compiler_params=pltpu.CompilerParams(
            dimension_semantics=("arbitrary", "arbitrary")),
    )(f1p, f2p, gid_col, *args)


def kernel(x, edge_index, edge_attr, graph_ids, params):
    p = params
    src = edge_index[0]
    dst = edge_index[1]
    zeros = jnp.zeros((ZROWS, D_NODE), jnp.float32)

    h = tc_node_proj(x, p['W_proj'], p['b_proj'].reshape(1, -1))
    z, ee = tc_edge_prep(edge_attr, p['We1'], p['be1'].reshape(1, -1),
                         p['Wpe'], p['bpe'].reshape(1, -1))

    eye = jnp.eye(D_NODE, dtype=jnp.float32)
    rmat = jnp.repeat(eye, D_NODE, axis=1)
    smat = jnp.tile(eye, (D_NODE, 1))
    node = h
    hidden = h
    for _ in range(STEPS):
        g = sc_gather(node, src)
        msg = tc_msg(z, g, p['We2'], p['be2'].reshape(1, -1), rmat, smat)
        aggp = sc_scatter_add(msg, dst, zeros)
        node, hidden = tc_gru(aggp, p['b_conv'].reshape(1, -1), node, hidden,
                              p['Wih_gru'], p['Whh_gru'],
                              p['bih_gru'].reshape(1, -1),
                              p['bhh_gru'].reshape(1, -1))

    f1p, f2p = sc_feat(node, ee, src, dst, zeros)

    wf3 = p['Wf3'].reshape(-1, N_TASKS, N_CLASSES)
    bf3 = p['bf3'].reshape(N_TASKS, N_CLASSES)
    tail_args = (
        p['lstm0_Wih'], p['lstm0_Whh'],
        (p['lstm0_bih'] + p['lstm0_bhh']).reshape(1, -1),
        p['lstm1_Wih'], p['lstm1_Whh'],
        (p['lstm1_bih'] + p['lstm1_bhh']).reshape(1, -1),
        p['lstm2_Wih'], p['lstm2_Whh'],
        (p['lstm2_bih'] + p['lstm2_bhh']).reshape(1, -1),
        p['Wf1'], p['bf1'].reshape(1, -1),
        p['Wf2'], p['bf2'].reshape(1, -1),
        wf3[:, :, 0], wf3[:, :, 1],
        bf3[:, 0].reshape(1, -1), bf3[:, 1].reshape(1, -1),
    )
    p0, p1 = tc_tail(f1p, f2p, graph_ids.reshape(-1, 1), tail_args)
    return jnp.stack([p0, p1], axis=-1)
